# baseline (speedup 1.0000x reference)
import functools

import jax
import jax.numpy as jnp
from jax import lax
from jax.experimental import pallas as pl
from jax.experimental.pallas import tpu as pltpu
from jax.experimental.pallas import tpu_sc as plsc

BATCH = 16384
EMBED_DIM = 64
NUM_CORES = 2
NUM_SUBCORES = 16
NUM_TILES = NUM_CORES * NUM_SUBCORES
ROWS_PER_TILE = BATCH // NUM_TILES

@functools.cache
def _sc_gather2():
    mesh = plsc.VectorSubcoreMesh(core_axis_name="c", subcore_axis_name="s")

    @functools.partial(
        pl.kernel,
        mesh=mesh,
        out_type=[
            jax.ShapeDtypeStruct((BATCH, EMBED_DIM), jnp.float32),
            jax.ShapeDtypeStruct((BATCH, EMBED_DIM), jnp.float32),
        ],
        scratch_types=[
            pltpu.SMEM((ROWS_PER_TILE,), jnp.int32),
            pltpu.VMEM((ROWS_PER_TILE,), jnp.int32),
            pltpu.VMEM((ROWS_PER_TILE, EMBED_DIM), jnp.float32),
            pltpu.SemaphoreType.DMA,
        ],
    )
    def gather2(uemb_hbm, memb_hbm, uid_hbm, mid_hbm, uout_hbm, mout_hbm,
                idx_s, idx_v, rows_v, sem):
        wid = lax.axis_index("s") * NUM_CORES + lax.axis_index("c")
        base = wid * ROWS_PER_TILE

        def one_table(tab_hbm, ids_hbm, out_hbm):
            pltpu.async_copy(ids_hbm.at[pl.ds(base, ROWS_PER_TILE)], idx_v,
                             sem).wait()

            @pl.loop(0, ROWS_PER_TILE, step=16)
            def _fire(c):
                vec = idx_v[pl.ds(c, 16)]
                for k in range(16):
                    pltpu.async_copy(tab_hbm.at[pl.ds(vec[k], 1)],
                                     rows_v.at[pl.ds(c + k, 1)], sem)

            @pl.loop(0, ROWS_PER_TILE)
            def _drain(j):
                pltpu.make_async_copy(tab_hbm.at[pl.ds(0, 1)],
                                      rows_v.at[pl.ds(0, 1)], sem).wait()

            pltpu.sync_copy(rows_v, out_hbm.at[pl.ds(base, ROWS_PER_TILE)])

        one_table(uemb_hbm, uid_hbm, uout_hbm)
        one_table(memb_hbm, mid_hbm, mout_hbm)

    return gather2


_BB = 2048


def _mlp_body(u_ref, m_ref, w1_ref, b1_ref, w2_ref, b2_ref, w3_ref, b3_ref,
              o_ref):
    f32 = jnp.float32
    hi = lax.Precision.HIGHEST
    dims = (((1,), (1,)), ((), ()))
    h = lax.dot_general(u_ref[...], w1_ref[:, :EMBED_DIM], dims,
                        precision=hi, preferred_element_type=f32)
    h += lax.dot_general(m_ref[...], w1_ref[:, EMBED_DIM:], dims,
                         precision=hi, preferred_element_type=f32)
    h = jnp.maximum(h + b1_ref[...], 0.0)
    h = lax.dot_general(h, w2_ref[...], dims, precision=hi,
                        preferred_element_type=f32)
    h = jnp.maximum(h + b2_ref[...], 0.0)
    o = jnp.sum(h * w3_ref[...], axis=1, keepdims=True)
    o_ref[...] = o + b3_ref[0]


def _mlp(user_vec, movie_vec, W1, b1, W2, b2, W3, b3):
    grid = (BATCH // _BB,)
    full = lambda *_: tuple(0 for _ in range(2))
    return pl.pallas_call(
        _mlp_body,
        grid=grid,
        in_specs=[
            pl.BlockSpec((_BB, EMBED_DIM), lambda i: (i, 0)),
            pl.BlockSpec((_BB, EMBED_DIM), lambda i: (i, 0)),
            pl.BlockSpec(W1.shape, full),
            pl.BlockSpec((1, 128), full),
            pl.BlockSpec(W2.shape, full),
            pl.BlockSpec((1, 64), full),
            pl.BlockSpec(W3.shape, full),
            pl.BlockSpec(memory_space=pltpu.SMEM),
        ],
        out_specs=pl.BlockSpec((_BB, 1), lambda i: (i, 0)),
        out_shape=jax.ShapeDtypeStruct((BATCH, 1), jnp.float32),
    )(user_vec, movie_vec, W1, b1, W2, b2, W3, b3)


def kernel(user_ids, movie_ids, user_emb, movie_emb, W1, b1, W2, b2, W3, b3):
    user_vec, movie_vec = _sc_gather2()(user_emb, movie_emb, user_ids,
                                        movie_ids)
    out = _mlp(user_vec, movie_vec, W1,
               b1.reshape(1, 128), W2, b2.reshape(1, 64), W3, b3)
    return jnp.squeeze(out, axis=1)
